# baseline (speedup 1.0000x reference)
import functools

import jax
import jax.numpy as jnp
from jax.experimental import pallas as pl
from jax.experimental.pallas import tpu as pltpu

_NH = 16
_BH = 512
_NC = 8
_NKH = 2


def _swiglu_body(x_ref, wg_ref, wu_ref, wd_ref, o_ref, z_ref):
    t = pl.program_id(1)

    @pl.when(t < _NH)
    def _phase1():
        xb = x_ref[...].astype(jnp.bfloat16)
        gate = jax.lax.dot_general(
            xb, wg_ref[...].astype(jnp.bfloat16), (((1,), (1,)), ((), ())),
            preferred_element_type=jnp.float32)
        up = jax.lax.dot_general(
            xb, wu_ref[...].astype(jnp.bfloat16), (((1,), (1,)), ((), ())),
            preferred_element_type=jnp.float32)
        z_ref[:, pl.ds(t * _BH, _BH)] = (
            gate * jax.nn.sigmoid(gate) * up).astype(jnp.bfloat16)

    @pl.when(t >= _NH)
    def _phase2():
        kh = (t - _NH) % _NKH
        khw = z_ref.shape[1] // _NKH
        zpart = z_ref[:, pl.ds(kh * khw, khw)]
        contrib = jax.lax.dot_general(
            zpart, wd_ref[...].astype(jnp.bfloat16), (((1,), (1,)), ((), ())),
            preferred_element_type=jnp.float32)

        @pl.when(kh == 0)
        def _first():
            o_ref[...] = contrib

        @pl.when(kh != 0)
        def _rest():
            o_ref[...] += contrib


@functools.partial(jax.jit, static_argnames=("bm",))
def _swiglu(x2d, Wg, Wu, Wd, bm=1024):
    m, d = x2d.shape
    hidden = Wg.shape[0]
    cw = d // _NC
    khw = hidden // _NKH
    grid = (m // bm, _NH + _NC * _NKH)

    def wgu_map(i, t):
        return (jnp.minimum(t, _NH - 1), 0)

    def wd_map(i, t):
        tt = jnp.maximum(t - _NH, 0)
        return (tt // _NKH, tt % _NKH)

    def o_map(i, t):
        return (i, jnp.maximum(t - _NH, 0) // _NKH)

    return pl.pallas_call(
        _swiglu_body,
        grid=grid,
        in_specs=[
            pl.BlockSpec((bm, d), lambda i, t: (i, 0),
                         pipeline_mode=pl.Buffered(buffer_count=1)),
            pl.BlockSpec((_BH, d), wgu_map),
            pl.BlockSpec((_BH, d), wgu_map),
            pl.BlockSpec((cw, khw), wd_map),
        ],
        out_specs=pl.BlockSpec((bm, cw), o_map),
        out_shape=jax.ShapeDtypeStruct((m, d), jnp.float32),
        scratch_shapes=[
            pltpu.VMEM((bm, hidden), jnp.bfloat16),
        ],
        compiler_params=pltpu.CompilerParams(
            dimension_semantics=("arbitrary", "arbitrary"),
        ),
    )(x2d, Wg, Wu, Wd)


def kernel(x, Wg, Wu, Wd):
    shape = x.shape
    d_model = shape[-1]
    x2d = x.reshape(-1, d_model)
    out = _swiglu(x2d, Wg, Wu, Wd)
    return out.reshape(shape)
